# baseline (speedup 1.0000x reference)
import functools

import jax
import jax.numpy as jnp
from jax import lax
from jax.experimental import pallas as pl
from jax.experimental.pallas import tpu as pltpu
from jax.experimental.pallas import tpu_sc as plsc

DIM = 768
MAXPOS = 512
EPS = 1e-12

L = 16
NV = DIM // L
NC = 2
NS = 16
NW = NC * NS
C = 64


def _rsqrt(v):
    ib = lax.bitcast_convert_type(v, jnp.int32)
    ib = jnp.int32(0x5F3759DF) - lax.shift_right_arithmetic(ib, 1)
    y = lax.bitcast_convert_type(ib, jnp.float32)
    for _ in range(3):
        y = y * (jnp.float32(1.5) - jnp.float32(0.5) * v * y * y)
    return y


def _make_pt_kernel(n_tt, max_pos):
    n_rows = n_tt * max_pos
    rpw = n_rows // NW
    mesh = plsc.VectorSubcoreMesh(core_axis_name="c", subcore_axis_name="s")

    @functools.partial(
        pl.kernel,
        out_type=jax.ShapeDtypeStruct((n_rows, DIM), jnp.float32),
        mesh=mesh,
        compiler_params=pltpu.CompilerParams(needs_layout_passes=False),
        scratch_types=[
            pltpu.VMEM((rpw, DIM), jnp.float32),
            pltpu.VMEM((n_tt * DIM,), jnp.float32),
        ],
    )
    def pt_kernel(ttemb_hbm, pemb_hbm, pt_hbm, buf_v, ttc_v):
        cid = lax.axis_index("c")
        sid = lax.axis_index("s")
        wid = sid * NC + cid
        base = wid * rpw
        prow = lax.rem(base, max_pos)
        variant = base // max_pos
        pltpu.sync_copy(pemb_hbm.at[pl.ds(prow, rpw)], buf_v)
        pltpu.sync_copy(ttemb_hbm, ttc_v)

        def row_body(r, carry):
            for j in range(NV):
                sl = pl.ds(j * L, L)
                tt = ttc_v[pl.ds(variant * DIM + j * L, L)]
                buf_v[r, sl] = buf_v[r, sl] + tt
            return carry

        lax.fori_loop(0, rpw, row_body, 0)
        pltpu.sync_copy(buf_v, pt_hbm.at[pl.ds(base, rpw)])

    return pt_kernel


def _make_emb_kernel(n_tokens, seq_len):
    tpw = n_tokens // NW
    nchunk = tpw // C
    inv_dim = jnp.float32(1.0 / DIM)
    mesh = plsc.VectorSubcoreMesh(core_axis_name="c", subcore_axis_name="s")

    @functools.partial(
        pl.kernel,
        out_type=jax.ShapeDtypeStruct((n_tokens, DIM), jnp.float32),
        mesh=mesh,
        compiler_params=pltpu.CompilerParams(needs_layout_passes=False),
        scratch_types=[
            pltpu.VMEM((C,), jnp.int32),
            pltpu.VMEM((C,), jnp.int32),
            pltpu.VMEM((C,), jnp.int32),
            pltpu.VMEM((C, DIM), jnp.float32),
            pltpu.VMEM((C, DIM), jnp.float32),
            pltpu.VMEM((DIM,), jnp.float32),
            pltpu.VMEM((DIM,), jnp.float32),
            pltpu.SemaphoreType.DMA,
        ],
    )
    def emb_kernel(ids_hbm, tt_hbm, wemb_hbm, pt_hbm, lnw_hbm, lnb_hbm,
                   out_hbm,
                   idx_v, ttidx_v, ptidx_v, rows_v, ptrows_v,
                   lnw_v, lnb_v, sem):
        cid = lax.axis_index("c")
        sid = lax.axis_index("s")
        wid = sid * NC + cid
        base0 = wid * tpw

        pltpu.sync_copy(lnw_hbm, lnw_v)
        pltpu.sync_copy(lnb_hbm, lnb_v)
        lane = lax.iota(jnp.int32, L)

        def chunk_body(k, carry):
            base = base0 + k * C
            pstart = lax.rem(base, seq_len)
            pltpu.sync_copy(ids_hbm.at[pl.ds(base, C)], idx_v)
            pltpu.sync_copy(tt_hbm.at[pl.ds(base, C)], ttidx_v)
            for g in range(C // L):
                sl = pl.ds(g * L, L)
                ttv = ttidx_v[sl]
                ptidx_v[sl] = lane + (pstart + g * L) + seq_len * ttv
            cp1 = pltpu.async_copy(wemb_hbm.at[idx_v], rows_v, sem)
            cp2 = pltpu.async_copy(pt_hbm.at[ptidx_v], ptrows_v, sem)
            cp1.wait()
            cp2.wait()

            def token_body(i, carry2):
                s = jnp.zeros((L,), jnp.float32)
                s2 = jnp.zeros((L,), jnp.float32)
                for j in range(NV):
                    sl = pl.ds(j * L, L)
                    x = rows_v[i, sl] + ptrows_v[i, sl]
                    rows_v[i, sl] = x
                    s = s + x
                    s2 = s2 + x * x
                mean = jnp.sum(s) * inv_dim
                ex2 = jnp.sum(s2) * inv_dim
                var = ex2 - mean * mean
                rstd = _rsqrt(var + jnp.float32(EPS))
                for j in range(NV):
                    sl = pl.ds(j * L, L)
                    x = rows_v[i, sl]
                    rows_v[i, sl] = (x - mean) * rstd * lnw_v[sl] + lnb_v[sl]
                return carry2

            lax.fori_loop(0, C, token_body, 0)
            pltpu.sync_copy(rows_v, out_hbm.at[pl.ds(base, C)])
            return carry

        lax.fori_loop(0, nchunk, chunk_body, 0)

    return emb_kernel


def kernel(input_ids, token_type_ids, word_embeddings, token_type_embeddings,
           position_embeddings, ln_weight, ln_bias):
    batch, seq = input_ids.shape
    n = batch * seq
    n_tt = token_type_embeddings.shape[0]
    ids = input_ids.reshape(n).astype(jnp.int32)
    tts = token_type_ids.reshape(n).astype(jnp.int32)
    pt = _make_pt_kernel(n_tt, seq)(
        token_type_embeddings.astype(jnp.float32).reshape(n_tt * DIM),
        position_embeddings.astype(jnp.float32),
    )
    emb = _make_emb_kernel(n, seq)(
        ids, tts,
        word_embeddings.astype(jnp.float32),
        pt,
        ln_weight.astype(jnp.float32),
        ln_bias.astype(jnp.float32),
    )
    return emb.reshape(batch, seq, DIM)


# speedup vs baseline: 1.1847x; 1.1847x over previous
import functools

import jax
import jax.numpy as jnp
from jax import lax
from jax.experimental import pallas as pl
from jax.experimental.pallas import tpu as pltpu
from jax.experimental.pallas import tpu_sc as plsc

DIM = 768
MAXPOS = 512
EPS = 1e-12

L = 16
NV = DIM // L
NC = 2
NS = 16
NW = NC * NS
C = 32


def _rsqrt(v):
    ib = lax.bitcast_convert_type(v, jnp.int32)
    ib = jnp.int32(0x5F3759DF) - lax.shift_right_arithmetic(ib, 1)
    y = lax.bitcast_convert_type(ib, jnp.float32)
    for _ in range(3):
        y = y * (jnp.float32(1.5) - jnp.float32(0.5) * v * y * y)
    return y


def _make_pt_kernel(n_tt, max_pos):
    n_rows = n_tt * max_pos
    rpw = n_rows // NW
    mesh = plsc.VectorSubcoreMesh(core_axis_name="c", subcore_axis_name="s")

    @functools.partial(
        pl.kernel,
        out_type=jax.ShapeDtypeStruct((n_rows, DIM), jnp.float32),
        mesh=mesh,
        compiler_params=pltpu.CompilerParams(needs_layout_passes=False),
        scratch_types=[
            pltpu.VMEM((rpw, DIM), jnp.float32),
            pltpu.VMEM((n_tt * DIM,), jnp.float32),
        ],
    )
    def pt_kernel(ttemb_hbm, pemb_hbm, pt_hbm, buf_v, ttc_v):
        cid = lax.axis_index("c")
        sid = lax.axis_index("s")
        wid = sid * NC + cid
        base = wid * rpw
        prow = lax.rem(base, max_pos)
        variant = base // max_pos
        pltpu.sync_copy(pemb_hbm.at[pl.ds(prow, rpw)], buf_v)
        pltpu.sync_copy(ttemb_hbm, ttc_v)

        def row_body(r, carry):
            for j in range(NV):
                sl = pl.ds(j * L, L)
                tt = ttc_v[pl.ds(variant * DIM + j * L, L)]
                buf_v[r, sl] = buf_v[r, sl] + tt
            return carry

        lax.fori_loop(0, rpw, row_body, 0)
        pltpu.sync_copy(buf_v, pt_hbm.at[pl.ds(base, rpw)])

    return pt_kernel


def _make_emb_kernel(n_tokens, seq_len):
    tpw = n_tokens // NW
    nchunk = tpw // C
    npair = nchunk // 2
    inv_dim = jnp.float32(1.0 / DIM)
    mesh = plsc.VectorSubcoreMesh(core_axis_name="c", subcore_axis_name="s")

    @functools.partial(
        pl.kernel,
        out_type=jax.ShapeDtypeStruct((n_tokens, DIM), jnp.float32),
        mesh=mesh,
        compiler_params=pltpu.CompilerParams(needs_layout_passes=False),
        scratch_types=[
            pltpu.VMEM((tpw,), jnp.int32),
            pltpu.VMEM((tpw,), jnp.int32),
            pltpu.VMEM((tpw,), jnp.int32),
            pltpu.VMEM((C, DIM), jnp.float32),
            pltpu.VMEM((C, DIM), jnp.float32),
            pltpu.VMEM((C, DIM), jnp.float32),
            pltpu.VMEM((C, DIM), jnp.float32),
            pltpu.VMEM((DIM,), jnp.float32),
            pltpu.VMEM((DIM,), jnp.float32),
            pltpu.SemaphoreType.DMA,
            pltpu.SemaphoreType.DMA,
            pltpu.SemaphoreType.DMA,
            pltpu.SemaphoreType.DMA,
        ],
    )
    def emb_kernel(ids_hbm, tt_hbm, wemb_hbm, pt_hbm, lnw_hbm, lnb_hbm,
                   out_hbm,
                   ids_v, tts_v, ptidx_v, rowsA, rowsB, ptA, ptB,
                   lnw_v, lnb_v, gsemA, gsemB, osemA, osemB):
        cid = lax.axis_index("c")
        sid = lax.axis_index("s")
        wid = sid * NC + cid
        base0 = wid * tpw

        pltpu.sync_copy(lnw_hbm, lnw_v)
        pltpu.sync_copy(lnb_hbm, lnb_v)
        pltpu.sync_copy(ids_hbm.at[pl.ds(base0, tpw)], ids_v)
        pltpu.sync_copy(tt_hbm.at[pl.ds(base0, tpw)], tts_v)
        lane = lax.iota(jnp.int32, L)
        for g in range(tpw // L):
            sl = pl.ds(g * L, L)
            ptidx_v[sl] = lane + ((g * L) % seq_len) + seq_len * tts_v[sl]

        def fire_gathers(k, rows_buf, pt_buf, gsem):
            sl = pl.ds(k * C, C)
            cpw = pltpu.async_copy(wemb_hbm.at[ids_v.at[sl]], rows_buf, gsem)
            cpp = pltpu.async_copy(pt_hbm.at[ptidx_v.at[sl]], pt_buf, gsem)
            return cpw, cpp

        def wait_gathers(k, rows_buf, pt_buf, gsem):
            sl = pl.ds(k * C, C)
            pltpu.make_async_copy(wemb_hbm.at[ids_v.at[sl]], rows_buf, gsem).wait()
            pltpu.make_async_copy(pt_hbm.at[ptidx_v.at[sl]], pt_buf, gsem).wait()

        def fire_out(k, rows_buf, osem):
            return pltpu.async_copy(rows_buf, out_hbm.at[pl.ds(base0 + k * C, C)], osem)

        def wait_out(k, rows_buf, osem):
            pltpu.make_async_copy(rows_buf, out_hbm.at[pl.ds(base0 + k * C, C)], osem).wait()

        def compute(rows_buf, pt_buf):
            def token_body(i, carry2):
                s = jnp.zeros((L,), jnp.float32)
                s2 = jnp.zeros((L,), jnp.float32)
                xs = []
                for j in range(NV):
                    sl = pl.ds(j * L, L)
                    x = rows_buf[i, sl] + pt_buf[i, sl]
                    xs.append(x)
                    s = s + x
                    s2 = s2 + x * x
                mean = jnp.sum(s) * inv_dim
                ex2 = jnp.sum(s2) * inv_dim
                var = ex2 - mean * mean
                rstd = _rsqrt(var + jnp.float32(EPS))
                shift = mean * rstd
                for j in range(NV):
                    sl = pl.ds(j * L, L)
                    xh = xs[j] * rstd - shift
                    rows_buf[i, sl] = xh * lnw_v[sl] + lnb_v[sl]
                return carry2

            lax.fori_loop(0, C, token_body, 0)

        fire_gathers(0, rowsA, ptA, gsemA)

        def pair_body(k2, carry):
            a = 2 * k2
            b = a + 1
            wait_gathers(a, rowsA, ptA, gsemA)

            @pl.when(k2 > 0)
            def _():
                wait_out(b - 2, rowsB, osemB)

            fire_gathers(b, rowsB, ptB, gsemB)
            compute(rowsA, ptA)
            fire_out(a, rowsA, osemA)

            wait_gathers(b, rowsB, ptB, gsemB)
            wait_out(a, rowsA, osemA)

            @pl.when(k2 < npair - 1)
            def _():
                fire_gathers(a + 2, rowsA, ptA, gsemA)

            compute(rowsB, ptB)
            fire_out(b, rowsB, osemB)
            return carry

        lax.fori_loop(0, npair, pair_body, 0)
        wait_out(nchunk - 1, rowsB, osemB)

    return emb_kernel


def kernel(input_ids, token_type_ids, word_embeddings, token_type_embeddings,
           position_embeddings, ln_weight, ln_bias):
    batch, seq = input_ids.shape
    n = batch * seq
    n_tt = token_type_embeddings.shape[0]
    ids = input_ids.reshape(n).astype(jnp.int32)
    tts = token_type_ids.reshape(n).astype(jnp.int32)
    pt = _make_pt_kernel(n_tt, seq)(
        token_type_embeddings.astype(jnp.float32).reshape(n_tt * DIM),
        position_embeddings.astype(jnp.float32),
    )
    emb = _make_emb_kernel(n, seq)(
        ids, tts,
        word_embeddings.astype(jnp.float32),
        pt,
        ln_weight.astype(jnp.float32),
        ln_bias.astype(jnp.float32),
    )
    return emb.reshape(batch, seq, DIM)
